# baseline (speedup 1.0000x reference)
import functools

import numpy as np
import jax
import jax.numpy as jnp
from jax import lax
from jax.experimental import pallas as pl
from jax.experimental.pallas import tpu as pltpu
from jax.experimental.pallas import tpu_sc as plsc

N = 10000
E = 160000
D = 256
H = 128
NS = 16
NC = 2
K = 128
EPAD = 163840
CPT_D = 80
NB_D = 2
NGRP_D = CPT_D // NB_D
EPAD_A = 165888
CPT_A = 81
NB_A = 3
NGRP_A = CPT_A // NB_A
PADID = 10008
NPAD = 10112
STRIPE = NPAD // NS

_mesh = plsc.VectorSubcoreMesh(core_axis_name="c", subcore_axis_name="s")


def _stage_idx(src2d, j, dstb):
    for u in range(8):
        dstb[pl.ds(16 * u, 16)] = src2d[j, pl.ds(16 * u, 16)]


@functools.partial(
    pl.kernel,
    out_type=[
        jax.ShapeDtypeStruct((NPAD, H), jnp.float32),
        jax.ShapeDtypeStruct((NPAD, H), jnp.float32),
    ],
    mesh=_mesh,
    scratch_types=[
        pltpu.VMEM_SHARED((NPAD, H), jnp.float32),
        pltpu.VMEM((CPT_D, K), jnp.int32),
        pltpu.VMEM((K,), jnp.int32),
        pltpu.VMEM((K,), jnp.int32),
        pltpu.VMEM((K, H), jnp.float32),
        pltpu.SemaphoreType.DMA,
        pltpu.SemaphoreType.DMA,
    ],
)
def _deg_kernel(src_hbm, dst_hbm, zrow, ones_hbm, deg_out, deg_in,
                dacc, idx_all, ib0, ib1, onesb, s0, s1):
    c = lax.axis_index("c")
    s = lax.axis_index("s")
    ibs = (ib0, ib1)
    sems = (s0, s1)
    stripe = s * jnp.int32(STRIPE)
    pltpu.sync_copy(zrow, dacc.at[pl.ds(stripe, STRIPE)])
    pltpu.sync_copy(ones_hbm, onesb)

    @pl.when(c == 0)
    def _():
        pltpu.sync_copy(src_hbm.at[s], idx_all)

    @pl.when(c == 1)
    def _():
        pltpu.sync_copy(dst_hbm.at[s], idx_all)

    plsc.subcore_barrier()

    for b in range(NB_D):
        _stage_idx(idx_all, jnp.int32(b), ibs[b])
        pltpu.async_copy(onesb, dacc.at[ibs[b]], sems[b], add=True)

    @pl.loop(jnp.int32(0), jnp.int32(NGRP_D))
    def grp(g):
        for b in range(NB_D):
            pltpu.make_async_copy(onesb, dacc.at[pl.ds(0, K)], sems[b]).wait()

            @pl.when(g < jnp.int32(NGRP_D - 1))
            def _():
                jn = g * jnp.int32(NB_D) + jnp.int32(b + NB_D)
                _stage_idx(idx_all, jn, ibs[b])
                pltpu.async_copy(onesb, dacc.at[ibs[b]], sems[b], add=True)

    plsc.subcore_barrier()

    @pl.when(c == 0)
    def _():
        pltpu.sync_copy(dacc.at[pl.ds(stripe, STRIPE)],
                        deg_out.at[pl.ds(stripe, STRIPE)])

    @pl.when(c == 1)
    def _():
        pltpu.sync_copy(dacc.at[pl.ds(stripe, STRIPE)],
                        deg_in.at[pl.ds(stripe, STRIPE)])


@functools.partial(
    pl.kernel,
    out_type=[
        jax.ShapeDtypeStruct((NPAD, H), jnp.float32),
        jax.ShapeDtypeStruct((NPAD, H), jnp.float32),
    ],
    mesh=_mesh,
    scratch_types=[
        pltpu.VMEM_SHARED((NPAD, H), jnp.float32),
        pltpu.VMEM((K,), jnp.int32),
        pltpu.VMEM((K,), jnp.int32),
        pltpu.VMEM((K,), jnp.int32),
        pltpu.VMEM((K,), jnp.int32),
        pltpu.VMEM((K,), jnp.int32),
        pltpu.VMEM((K,), jnp.int32),
        pltpu.VMEM((K, H), jnp.float32),
        pltpu.VMEM((K, H), jnp.float32),
        pltpu.VMEM((K, H), jnp.float32),
        pltpu.SemaphoreType.DMA,
        pltpu.SemaphoreType.DMA,
        pltpu.SemaphoreType.DMA,
        pltpu.SemaphoreType.DMA,
        pltpu.SemaphoreType.DMA,
        pltpu.SemaphoreType.DMA,
        pltpu.SemaphoreType.DMA,
        pltpu.SemaphoreType.DMA,
        pltpu.SemaphoreType.DMA,
        pltpu.SemaphoreType.DMA,
        pltpu.SemaphoreType.DMA,
        pltpu.SemaphoreType.DMA,
    ],
)
def _agg_kernel(hna, hnb, src_hbm, dst_hbm, zrow, agga, aggb,
                acc, db0, db1, db2, sb0, sb1, sb2,
                r0, r1, r2, g0, g1, g2, t0, t1, t2,
                u0, u1, u2, v0, v1, v2):
    c = lax.axis_index("c")
    s = lax.axis_index("s")
    dbs = (db0, db1, db2)
    sbs = (sb0, sb1, sb2)
    rows = (r0, r1, r2)
    gs = (g0, g1, g2)
    ss = (t0, t1, t2)
    iS = (u0, u1, u2)
    iD = (v0, v1, v2)
    stripe = s * jnp.int32(STRIPE)
    ebase = s * jnp.int32(CPT_A * K)
    pltpu.sync_copy(zrow, acc.at[pl.ds(stripe, STRIPE)])
    plsc.subcore_barrier()

    def pipe(hn):
        for b in range(NB_A):
            pltpu.async_copy(src_hbm.at[pl.ds(ebase + jnp.int32(b * K), K)],
                             sbs[b], iS[b])
            pltpu.async_copy(dst_hbm.at[pl.ds(ebase + jnp.int32(b * K), K)],
                             dbs[b], iD[b])
        for b in range(NB_A):
            pltpu.make_async_copy(src_hbm.at[pl.ds(0, K)], sbs[b], iS[b]).wait()
            pltpu.async_copy(hn.at[sbs[b]], rows[b], gs[b])

        @pl.loop(jnp.int32(0), jnp.int32(NGRP_A))
        def grp(g):
            for b in range(NB_A):
                j = g * jnp.int32(NB_A) + jnp.int32(b)
                pltpu.make_async_copy(hn.at[pl.ds(0, K)], rows[b], gs[b]).wait()
                pltpu.make_async_copy(dst_hbm.at[pl.ds(0, K)], dbs[b], iD[b]).wait()
                pltpu.async_copy(rows[b], acc.at[dbs[b]], ss[b], add=True)

                @pl.when(g < jnp.int32(NGRP_A - 1))
                def _():
                    pltpu.async_copy(
                        src_hbm.at[pl.ds(ebase + (j + jnp.int32(NB_A)) * jnp.int32(K), K)],
                        sbs[b], iS[b])
            for b in range(NB_A):
                j = g * jnp.int32(NB_A) + jnp.int32(b)
                pltpu.make_async_copy(rows[b], acc.at[pl.ds(0, K)], ss[b]).wait()

                @pl.when(g < jnp.int32(NGRP_A - 1))
                def _():
                    pltpu.async_copy(
                        dst_hbm.at[pl.ds(ebase + (j + jnp.int32(NB_A)) * jnp.int32(K), K)],
                        dbs[b], iD[b])
                    pltpu.make_async_copy(src_hbm.at[pl.ds(0, K)], sbs[b], iS[b]).wait()
                    pltpu.async_copy(hn.at[sbs[b]], rows[b], gs[b])

    @pl.when(c == 0)
    def _():
        pipe(hna)

    @pl.when(c == 1)
    def _():
        pipe(hnb)

    plsc.subcore_barrier()

    @pl.when(c == 0)
    def _():
        pltpu.sync_copy(acc.at[pl.ds(stripe, STRIPE)],
                        agga.at[pl.ds(stripe, STRIPE)])

    @pl.when(c == 1)
    def _():
        pltpu.sync_copy(acc.at[pl.ds(stripe, STRIPE)],
                        aggb.at[pl.ds(stripe, STRIPE)])


_RB = 1000
_I0 = np.int32(0)


def _t1_body(x_ref, dego_ref, ha_ref, hb_ref):
    d = dego_ref[:, 0:1]
    sc = lax.rsqrt(jnp.maximum(d, 1.0))
    xv = x_ref[...]
    ha_ref[...] = xv[:, :H] * sc
    hb_ref[...] = xv[:, H:] * sc


_t1 = pl.pallas_call(
    _t1_body,
    grid=(N // _RB,),
    in_specs=[
        pl.BlockSpec((_RB, D), lambda i: (i, _I0)),
        pl.BlockSpec((_RB, H), lambda i: (i, _I0)),
    ],
    out_specs=[
        pl.BlockSpec((_RB, H), lambda i: (i, _I0)),
        pl.BlockSpec((_RB, H), lambda i: (i, _I0)),
    ],
    out_shape=[
        jax.ShapeDtypeStruct((N, H), jnp.float32),
        jax.ShapeDtypeStruct((N, H), jnp.float32),
    ],
)


def _t2_body(aa_ref, ab_ref, degi_ref, dego_ref, w_ref, b_ref, oa_ref, ob_ref):
    si = lax.rsqrt(jnp.maximum(degi_ref[:, 0:1], 1.0))
    so = lax.rsqrt(jnp.maximum(dego_ref[:, 0:1], 1.0))
    h = (jnp.dot(aa_ref[...] * si, w_ref[:H, :],
                 preferred_element_type=jnp.float32)
         + jnp.dot(ab_ref[...] * si, w_ref[H:, :],
                   preferred_element_type=jnp.float32)
         + b_ref[...])
    h = jnp.maximum(h, 0.0) * so
    oa_ref[...] = h[:, :H]
    ob_ref[...] = h[:, H:]


_t2 = pl.pallas_call(
    _t2_body,
    grid=(N // _RB,),
    in_specs=[
        pl.BlockSpec((_RB, H), lambda i: (i, _I0)),
        pl.BlockSpec((_RB, H), lambda i: (i, _I0)),
        pl.BlockSpec((_RB, H), lambda i: (i, _I0)),
        pl.BlockSpec((_RB, H), lambda i: (i, _I0)),
        pl.BlockSpec((D, D), lambda i: (_I0, _I0)),
        pl.BlockSpec((1, D), lambda i: (_I0, _I0)),
    ],
    out_specs=[
        pl.BlockSpec((_RB, H), lambda i: (i, _I0)),
        pl.BlockSpec((_RB, H), lambda i: (i, _I0)),
    ],
    out_shape=[
        jax.ShapeDtypeStruct((N, H), jnp.float32),
        jax.ShapeDtypeStruct((N, H), jnp.float32),
    ],
)


def _t3_body(aa_ref, ab_ref, degi_ref, w_ref, b_ref, o_ref):
    si = lax.rsqrt(jnp.maximum(degi_ref[:, 0:1], 1.0))
    o_ref[...] = (jnp.dot(aa_ref[...] * si, w_ref[:H, :],
                          preferred_element_type=jnp.float32)
                  + jnp.dot(ab_ref[...] * si, w_ref[H:, :],
                            preferred_element_type=jnp.float32)
                  + b_ref[...])


_t3 = pl.pallas_call(
    _t3_body,
    grid=(N // _RB,),
    in_specs=[
        pl.BlockSpec((_RB, H), lambda i: (i, _I0)),
        pl.BlockSpec((_RB, H), lambda i: (i, _I0)),
        pl.BlockSpec((_RB, H), lambda i: (i, _I0)),
        pl.BlockSpec((D, D), lambda i: (_I0, _I0)),
        pl.BlockSpec((1, D), lambda i: (_I0, _I0)),
    ],
    out_specs=pl.BlockSpec((_RB, D), lambda i: (i, _I0)),
    out_shape=jax.ShapeDtypeStruct((N, D), jnp.float32),
)


def kernel(x, edge_index, W1, b1, W2, b2):
    src = edge_index[0].astype(jnp.int32)
    dst = edge_index[1].astype(jnp.int32)
    x = x.astype(jnp.float32)
    W1 = W1.astype(jnp.float32)
    W2 = W2.astype(jnp.float32)
    b1r = b1.astype(jnp.float32).reshape(1, D)
    b2r = b2.astype(jnp.float32).reshape(1, D)

    npadD = EPAD - E
    padvD = jnp.full((npadD,), PADID, jnp.int32)
    dst_p = jnp.concatenate([dst, padvD]).reshape(NS, CPT_D, K)
    src_p = jnp.concatenate([src, padvD]).reshape(NS, CPT_D, K)
    npadA = EPAD_A - E
    src_a = jnp.concatenate([src, jnp.zeros((npadA,), jnp.int32)])
    dst_a = jnp.concatenate([dst, jnp.full((npadA,), PADID, jnp.int32)])

    zrow = jnp.zeros((STRIPE, H), jnp.float32)
    ones128 = jnp.ones((K, H), jnp.float32)

    deg_out, deg_in = _deg_kernel(src_p, dst_p, zrow, ones128)
    deg_out = deg_out[:N]
    deg_in = deg_in[:N]

    hna, hnb = _t1(x, deg_out)
    agga, aggb = _agg_kernel(hna, hnb, src_a, dst_a, zrow)
    hna2, hnb2 = _t2(agga[:N], aggb[:N], deg_in, deg_out, W1, b1r)
    agga2, aggb2 = _agg_kernel(hna2, hnb2, src_a, dst_a, zrow)
    return _t3(agga2[:N], aggb2[:N], deg_in, W2, b2r).astype(jnp.float64)


# speedup vs baseline: 1.3800x; 1.3800x over previous
import functools

import numpy as np
import jax
import jax.numpy as jnp
from jax import lax
from jax.experimental import pallas as pl
from jax.experimental.pallas import tpu as pltpu
from jax.experimental.pallas import tpu_sc as plsc

N = 10000
E = 160000
D = 256
H = 128
NS = 16
NC = 2
K = 128
EPAD = 163840
CPT = 80
NB = 2
NGRP = CPT // NB
PADID = 10008
NPAD = 10240
STRIPE = NPAD // NS

_mesh = plsc.VectorSubcoreMesh(core_axis_name="c", subcore_axis_name="s")


def _stage_idx(src2d, j, dstb):
    for u in range(8):
        dstb[pl.ds(16 * u, 16)] = src2d[j, pl.ds(16 * u, 16)]


@functools.partial(
    pl.kernel,
    out_type=[
        jax.ShapeDtypeStruct((NPAD, H), jnp.float32),
        jax.ShapeDtypeStruct((NPAD, H), jnp.float32),
    ],
    mesh=_mesh,
    scratch_types=[
        pltpu.VMEM_SHARED((NPAD, H), jnp.float32),
        pltpu.VMEM((CPT, K), jnp.int32),
        pltpu.VMEM((K,), jnp.int32),
        pltpu.VMEM((K,), jnp.int32),
        pltpu.VMEM((K, H), jnp.float32),
        pltpu.SemaphoreType.DMA,
        pltpu.SemaphoreType.DMA,
    ],
)
def _deg_kernel(src_hbm, dst_hbm, zrow, ones_hbm, deg_out, deg_in,
                dacc, idx_all, ib0, ib1, onesb, s0, s1):
    c = lax.axis_index("c")
    s = lax.axis_index("s")
    ibs = (ib0, ib1)
    sems = (s0, s1)
    stripe = s * jnp.int32(STRIPE)
    pltpu.sync_copy(zrow, dacc.at[pl.ds(stripe, STRIPE)])
    pltpu.sync_copy(ones_hbm, onesb)

    @pl.when(c == 0)
    def _():
        pltpu.sync_copy(src_hbm.at[s], idx_all)

    @pl.when(c == 1)
    def _():
        pltpu.sync_copy(dst_hbm.at[s], idx_all)

    plsc.subcore_barrier()

    for b in range(NB):
        _stage_idx(idx_all, jnp.int32(b), ibs[b])
        pltpu.async_copy(onesb, dacc.at[ibs[b]], sems[b], add=True)

    @pl.loop(jnp.int32(0), jnp.int32(NGRP))
    def grp(g):
        for b in range(NB):
            pltpu.make_async_copy(onesb, dacc.at[pl.ds(0, K)], sems[b]).wait()

            @pl.when(g < jnp.int32(NGRP - 1))
            def _():
                jn = g * jnp.int32(NB) + jnp.int32(b + NB)
                _stage_idx(idx_all, jn, ibs[b])
                pltpu.async_copy(onesb, dacc.at[ibs[b]], sems[b], add=True)

    plsc.subcore_barrier()

    @pl.when(c == 0)
    def _():
        pltpu.sync_copy(dacc.at[pl.ds(stripe, STRIPE)],
                        deg_out.at[pl.ds(stripe, STRIPE)])

    @pl.when(c == 1)
    def _():
        pltpu.sync_copy(dacc.at[pl.ds(stripe, STRIPE)],
                        deg_in.at[pl.ds(stripe, STRIPE)])


@functools.partial(
    pl.kernel,
    out_type=[
        jax.ShapeDtypeStruct((NPAD, H), jnp.float32),
        jax.ShapeDtypeStruct((NPAD, H), jnp.float32),
    ],
    mesh=_mesh,
    scratch_types=[
        pltpu.VMEM_SHARED((NPAD, H), jnp.float32),
        pltpu.VMEM((CPT, K), jnp.int32),
        pltpu.VMEM((K,), jnp.int32),
        pltpu.VMEM((K,), jnp.int32),
        pltpu.VMEM((K,), jnp.int32),
        pltpu.VMEM((K,), jnp.int32),
        pltpu.VMEM((K, H), jnp.float32),
        pltpu.VMEM((K, H), jnp.float32),
        pltpu.SemaphoreType.DMA,
        pltpu.SemaphoreType.DMA,
        pltpu.SemaphoreType.DMA,
        pltpu.SemaphoreType.DMA,
        pltpu.SemaphoreType.DMA,
        pltpu.SemaphoreType.DMA,
    ],
)
def _agg_kernel(hna, hnb, src_hbm, dst_hbm, zrow, agga, aggb,
                acc, idxd_all, ib0, ib1, sb0, sb1,
                r0, r1, g0, g1, t0, t1, i0, i1):
    c = lax.axis_index("c")
    s = lax.axis_index("s")
    ibs = (ib0, ib1)
    sbs = (sb0, sb1)
    rows = (r0, r1)
    gs = (g0, g1)
    ss = (t0, t1)
    isems = (i0, i1)
    stripe = s * jnp.int32(STRIPE)
    ebase = s * jnp.int32(CPT * K)
    pltpu.sync_copy(zrow, acc.at[pl.ds(stripe, STRIPE)])
    pltpu.sync_copy(dst_hbm.at[s], idxd_all)
    plsc.subcore_barrier()

    def pipe(hn):
        for b in range(NB):
            pltpu.async_copy(src_hbm.at[pl.ds(ebase + jnp.int32(b * K), K)],
                             sbs[b], isems[b])
        for b in range(NB):
            pltpu.make_async_copy(src_hbm.at[pl.ds(0, K)], sbs[b], isems[b]).wait()
            pltpu.async_copy(hn.at[sbs[b]], rows[b], gs[b])

        @pl.loop(jnp.int32(0), jnp.int32(NGRP))
        def grp(g):
            for b in range(NB):
                j = g * jnp.int32(NB) + jnp.int32(b)
                pltpu.make_async_copy(hn.at[pl.ds(0, K)], rows[b], gs[b]).wait()
                _stage_idx(idxd_all, j, ibs[b])
                pltpu.async_copy(rows[b], acc.at[ibs[b]], ss[b], add=True)

                @pl.when(g < jnp.int32(NGRP - 1))
                def _():
                    pltpu.async_copy(
                        src_hbm.at[pl.ds(ebase + (j + jnp.int32(NB)) * jnp.int32(K), K)],
                        sbs[b], isems[b])
            for b in range(NB):
                pltpu.make_async_copy(rows[b], acc.at[pl.ds(0, K)], ss[b]).wait()

                @pl.when(g < jnp.int32(NGRP - 1))
                def _():
                    pltpu.make_async_copy(src_hbm.at[pl.ds(0, K)], sbs[b], isems[b]).wait()
                    pltpu.async_copy(hn.at[sbs[b]], rows[b], gs[b])

    @pl.when(c == 0)
    def _():
        pipe(hna)

    @pl.when(c == 1)
    def _():
        pipe(hnb)

    plsc.subcore_barrier()

    @pl.when(c == 0)
    def _():
        pltpu.sync_copy(acc.at[pl.ds(stripe, STRIPE)],
                        agga.at[pl.ds(stripe, STRIPE)])

    @pl.when(c == 1)
    def _():
        pltpu.sync_copy(acc.at[pl.ds(stripe, STRIPE)],
                        aggb.at[pl.ds(stripe, STRIPE)])


_RB = 1000
_I0 = np.int32(0)


def _t1_body(x_ref, dego_ref, ha_ref, hb_ref):
    d = dego_ref[:, 0:1]
    sc = lax.rsqrt(jnp.maximum(d, 1.0))
    xv = x_ref[...]
    ha_ref[...] = xv[:, :H] * sc
    hb_ref[...] = xv[:, H:] * sc


_t1 = pl.pallas_call(
    _t1_body,
    grid=(N // _RB,),
    in_specs=[
        pl.BlockSpec((_RB, D), lambda i: (i, _I0)),
        pl.BlockSpec((_RB, H), lambda i: (i, _I0)),
    ],
    out_specs=[
        pl.BlockSpec((_RB, H), lambda i: (i, _I0)),
        pl.BlockSpec((_RB, H), lambda i: (i, _I0)),
    ],
    out_shape=[
        jax.ShapeDtypeStruct((N, H), jnp.float32),
        jax.ShapeDtypeStruct((N, H), jnp.float32),
    ],
)


def _t2_body(aa_ref, ab_ref, degi_ref, dego_ref, w_ref, b_ref, oa_ref, ob_ref):
    si = lax.rsqrt(jnp.maximum(degi_ref[:, 0:1], 1.0))
    so = lax.rsqrt(jnp.maximum(dego_ref[:, 0:1], 1.0))
    h = (jnp.dot(aa_ref[...] * si, w_ref[:H, :],
                 preferred_element_type=jnp.float32)
         + jnp.dot(ab_ref[...] * si, w_ref[H:, :],
                   preferred_element_type=jnp.float32)
         + b_ref[...])
    h = jnp.maximum(h, 0.0) * so
    oa_ref[...] = h[:, :H]
    ob_ref[...] = h[:, H:]


_t2 = pl.pallas_call(
    _t2_body,
    grid=(N // _RB,),
    in_specs=[
        pl.BlockSpec((_RB, H), lambda i: (i, _I0)),
        pl.BlockSpec((_RB, H), lambda i: (i, _I0)),
        pl.BlockSpec((_RB, H), lambda i: (i, _I0)),
        pl.BlockSpec((_RB, H), lambda i: (i, _I0)),
        pl.BlockSpec((D, D), lambda i: (_I0, _I0)),
        pl.BlockSpec((1, D), lambda i: (_I0, _I0)),
    ],
    out_specs=[
        pl.BlockSpec((_RB, H), lambda i: (i, _I0)),
        pl.BlockSpec((_RB, H), lambda i: (i, _I0)),
    ],
    out_shape=[
        jax.ShapeDtypeStruct((N, H), jnp.float32),
        jax.ShapeDtypeStruct((N, H), jnp.float32),
    ],
)


def _t3_body(aa_ref, ab_ref, degi_ref, w_ref, b_ref, o_ref):
    si = lax.rsqrt(jnp.maximum(degi_ref[:, 0:1], 1.0))
    o_ref[...] = (jnp.dot(aa_ref[...] * si, w_ref[:H, :],
                          preferred_element_type=jnp.float32)
                  + jnp.dot(ab_ref[...] * si, w_ref[H:, :],
                            preferred_element_type=jnp.float32)
                  + b_ref[...])


_t3 = pl.pallas_call(
    _t3_body,
    grid=(N // _RB,),
    in_specs=[
        pl.BlockSpec((_RB, H), lambda i: (i, _I0)),
        pl.BlockSpec((_RB, H), lambda i: (i, _I0)),
        pl.BlockSpec((_RB, H), lambda i: (i, _I0)),
        pl.BlockSpec((D, D), lambda i: (_I0, _I0)),
        pl.BlockSpec((1, D), lambda i: (_I0, _I0)),
    ],
    out_specs=pl.BlockSpec((_RB, D), lambda i: (i, _I0)),
    out_shape=jax.ShapeDtypeStruct((N, D), jnp.float32),
)


def kernel(x, edge_index, W1, b1, W2, b2):
    src = edge_index[0].astype(jnp.int32)
    dst = edge_index[1].astype(jnp.int32)
    x = x.astype(jnp.float32)
    W1 = W1.astype(jnp.float32)
    W2 = W2.astype(jnp.float32)
    b1r = b1.astype(jnp.float32).reshape(1, D)
    b2r = b2.astype(jnp.float32).reshape(1, D)

    npadE = EPAD - E
    padv = jnp.full((npadE,), PADID, jnp.int32)
    src_a = jnp.concatenate([src, jnp.zeros((npadE,), jnp.int32)])
    dst_p = jnp.concatenate([dst, padv]).reshape(NS, CPT, K)
    src_p = jnp.concatenate([src, padv]).reshape(NS, CPT, K)

    zrow = jnp.zeros((STRIPE, H), jnp.float32)
    ones128 = jnp.ones((K, H), jnp.float32)

    deg_out, deg_in = _deg_kernel(src_p, dst_p, zrow, ones128)

    hna, hnb = _t1(x, deg_out)
    agga, aggb = _agg_kernel(hna, hnb, src_a, dst_p, zrow)
    hna2, hnb2 = _t2(agga, aggb, deg_in, deg_out, W1, b1r)
    agga2, aggb2 = _agg_kernel(hna2, hnb2, src_a, dst_p, zrow)
    return _t3(agga2, aggb2, deg_in, W2, b2r).astype(jnp.float64)
